# Initial kernel scaffold; baseline (speedup 1.0000x reference)
import jax
import jax.numpy as jnp
from jax.experimental import pallas as pl


def kernel(nf, ef_cc, ef_ss, params, edge_cc, edge_ss):
    raise NotImplementedError("write your pallas kernel here")


# speedup vs baseline: 1.0059x; 1.0059x over previous
import functools

import jax
import jax.numpy as jnp
from jax.experimental import pallas as pl
from jax.experimental.pallas import tpu as pltpu

N = 10000
E = 160000
BE = 2000


def _lin(p, x):
    return x @ p["W"] + p["b"]


def _mlp(ps, x):
    for i, p in enumerate(ps):
        x = _lin(p, x)
        if i < len(ps) - 1:
            x = jax.nn.relu(x)
    return x


def _bn(p, x, eps=1e-5):
    m = jnp.mean(x, axis=0)
    v = jnp.var(x, axis=0)
    return (x - m) / jnp.sqrt(v + eps) * p["gamma"] + p["beta"]


def _mp_block(p, n_out, nf, ef, src, dst, n_nodes):
    x = jnp.concatenate([nf[src], nf[dst], ef], axis=1)
    x = _mlp(p["msg"], x)
    k = jax.nn.sigmoid(x[:, :1])
    f1 = x[:, 1:1 + n_out] * k
    f2 = x[:, 1 + n_out:1 + 2 * n_out] * k
    f3 = x[:, 1 + 2 * n_out:1 + 3 * n_out] * k
    f4 = x[:, 1 + 3 * n_out:1 + 4 * n_out] * k
    cnt = jax.ops.segment_sum(jnp.ones((ef.shape[0], 1), jnp.float32), dst, n_nodes)
    has = cnt > 0
    n1 = jax.ops.segment_sum(f1, dst, n_nodes)
    n2 = jnp.where(has, jax.ops.segment_max(f2, dst, n_nodes), 0.0)
    n3 = jnp.where(has, jax.ops.segment_min(f3, dst, n_nodes), 0.0)
    n4 = jax.ops.segment_sum(f4, dst, n_nodes) / jnp.maximum(cnt, 1.0)
    vc = _mlp(p["reduce"], jnp.concatenate([nf, n1, n2, n3, n4], axis=1))
    ve = _mlp(p["ve_down"], jnp.concatenate([f1, f2, f3, f4, ef], axis=1))
    return _bn(p["bn_nf"], vc), _bn(p["bn_ef"], ve)


def _toedge_body(x_ref, w1, b1, w2, b2, w3, b3, w4, b4, w5, b5, out_ref):
    x = x_ref[...]
    h = jnp.maximum(x @ w1[...] + b1[...], 0.0)
    h = jnp.maximum(h @ w2[...] + b2[...], 0.0)
    y = h @ w3[...] + b3[...]
    k = jax.nn.sigmoid(y[:, :1])
    f = y[:, 1:65] * k
    g = jnp.maximum(f @ w4[...] + b4[...], 0.0)
    out_ref[...] = g @ w5[...] + b5[...]


def _toedge_pallas(x, p):
    w1, b1 = p["mlp1"][0]["W"], p["mlp1"][0]["b"]
    w2, b2 = p["mlp1"][1]["W"], p["mlp1"][1]["b"]
    w3, b3 = p["mlp1"][2]["W"], p["mlp1"][2]["b"]
    w4, b4 = p["mlp2"][0]["W"], p["mlp2"][0]["b"]
    w5, b5 = p["mlp2"][1]["W"], p["mlp2"][1]["b"]
    grid = E // BE
    full = lambda a: pl.BlockSpec(a.shape, lambda i: (0,) * a.ndim)
    return pl.pallas_call(
        _toedge_body,
        grid=(grid,),
        in_specs=[pl.BlockSpec((BE, x.shape[1]), lambda i: (i, 0))]
        + [full(a) for a in (w1, b1, w2, b2, w3, b3, w4, b4, w5, b5)],
        out_specs=pl.BlockSpec((BE, 1), lambda i: (i, 0)),
        out_shape=jax.ShapeDtypeStruct((E, 1), jnp.float32),
    )(x, w1, b1, w2, b2, w3, b3, w4, b4, w5, b5)


def kernel(nf, ef_cc, ef_ss, params, edge_cc, edge_ss):
    n = nf.shape[0]
    nf = nf.at[:, 1:3].set(0.0)
    ef_cc = jnp.zeros_like(ef_cc)
    ss_src, ss_dst = edge_ss[0], edge_ss[1]
    cc_src, cc_dst = edge_cc[0], edge_cc[1]
    vc0 = _lin(params["fc_ss_nf"], nf)
    ve0 = _lin(params["fc_ss_ef"], ef_ss)
    vc1, ve1 = _mp_block(params["gsmp1"], 64, vc0, ve0, ss_src, ss_dst, n)
    vc2, ve2 = _mp_block(params["gsmp2"], 128, vc1, ve1, ss_src, ss_dst, n)
    vc3, ve3 = _mp_block(params["gsmp3"], 64,
                         jnp.concatenate([vc2, vc1], axis=1),
                         jnp.concatenate([ve2, ve1], axis=1), ss_src, ss_dst, n)
    vc4 = _mlp(params["mlp_ss_nf"], jnp.concatenate([vc3, vc0], axis=1))
    ve4 = _mlp(params["mlp_ss_ef"], jnp.concatenate([ve3, ve0], axis=1))
    del ve4
    cc_vc0 = _lin(params["fc_cc_nf"], jnp.concatenate([nf, vc4], axis=1))
    cc_ve0 = _lin(params["fc_cc_ef"], ef_cc)
    cc_vc1, cc_ve1 = _mp_block(params["gcmp1"], 64, cc_vc0, cc_ve0, cc_src, cc_dst, n)
    cc_vc2, cc_ve2 = _mp_block(params["gcmp2"], 128, cc_vc1, cc_ve1, cc_src, cc_dst, n)
    cc_vc3, cc_ve3 = _mp_block(params["gcmp3"], 64,
                               jnp.concatenate([cc_vc2, cc_vc1], axis=1),
                               jnp.concatenate([cc_ve2, cc_ve1], axis=1), cc_src, cc_dst, n)
    nf_t = jnp.concatenate([cc_vc3, cc_vc0], axis=1)
    ef_t = jnp.concatenate([cc_ve3, cc_ve0, ef_cc], axis=1)
    x = jnp.concatenate([nf_t[cc_src], nf_t[cc_dst], ef_t], axis=1)
    return _toedge_pallas(x, params["toedge"])


# speedup vs baseline: 1.2016x; 1.1946x over previous
import functools

import jax
import jax.numpy as jnp
from jax import lax
from jax.experimental import pallas as pl
from jax.experimental.pallas import tpu as pltpu
from jax.experimental.pallas import tpu_sc as plsc

N = 10000
E = 160000
BE = 2000

NC, NS = 2, 16
NW = NC * NS
CH = 128
NCHUNK = E // CH
NPAD = 10112
RPS = NPAD // NS


def _sc_scatter_add(vals, idx):
    D = vals.shape[1]
    assert D == 128, D
    mesh = plsc.VectorSubcoreMesh(core_axis_name="c", subcore_axis_name="s")

    @functools.partial(
        pl.kernel, mesh=mesh,
        out_type=jax.ShapeDtypeStruct((NC * NPAD, D), jnp.float32),
        scratch_types=[
            pltpu.VMEM((CH,), jnp.int32),
            pltpu.VMEM((CH, D), jnp.float32),
            pltpu.VMEM_SHARED((NPAD, D), jnp.float32),
        ],
    )
    def k(vals_hbm, idx_hbm, zeros_hbm, out_hbm, idx_v, vals_v, acc):
        c = lax.axis_index("c")
        s = lax.axis_index("s")
        g = c * NS + s
        pltpu.sync_copy(zeros_hbm, acc.at[pl.ds(s * RPS, RPS)])
        plsc.subcore_barrier()
        nchunks = jnp.where(g < NCHUNK % NW, NCHUNK // NW + 1, NCHUNK // NW)
        def body(t, _):
            cid = g + t * NW
            pltpu.sync_copy(idx_hbm.at[pl.ds(cid * CH, CH)], idx_v)
            pltpu.sync_copy(vals_hbm.at[pl.ds(cid * CH, CH)], vals_v)
            pltpu.sync_copy(vals_v, acc.at[idx_v], add=True)
            return 0
        lax.fori_loop(0, nchunks, body, 0)
        plsc.subcore_barrier()
        pltpu.sync_copy(acc.at[pl.ds(s * RPS, RPS)],
                        out_hbm.at[pl.ds(c * NPAD + s * RPS, RPS)])

    return k(vals, idx, jnp.zeros((RPS, D), jnp.float32))


def _segsum(vals, idx):
    part = _sc_scatter_add(vals, idx)
    return part[:N] + part[NPAD:NPAD + N]


def _lin(p, x):
    return x @ p["W"] + p["b"]


def _mlp(ps, x):
    for i, p in enumerate(ps):
        x = _lin(p, x)
        if i < len(ps) - 1:
            x = jax.nn.relu(x)
    return x


def _bn(p, x, eps=1e-5):
    m = jnp.mean(x, axis=0)
    v = jnp.var(x, axis=0)
    return (x - m) / jnp.sqrt(v + eps) * p["gamma"] + p["beta"]


def _mp_block(p, n_out, nf, ef, src, dst, n_nodes, cnt):
    x = jnp.concatenate([nf[src], nf[dst], ef], axis=1)
    x = _mlp(p["msg"], x)
    k = jax.nn.sigmoid(x[:, :1])
    f1 = x[:, 1:1 + n_out] * k
    f2 = x[:, 1 + n_out:1 + 2 * n_out] * k
    f3 = x[:, 1 + 2 * n_out:1 + 3 * n_out] * k
    f4 = x[:, 1 + 3 * n_out:1 + 4 * n_out] * k
    has = cnt > 0
    if n_out == 64:
        s14 = _segsum(jnp.concatenate([f1, f4], axis=1), dst)
        n1, n4s = s14[:, :64], s14[:, 64:]
    else:
        n1 = _segsum(f1, dst)
        n4s = _segsum(f4, dst)
    n2 = jnp.where(has, jax.ops.segment_max(f2, dst, n_nodes), 0.0)
    n3 = jnp.where(has, jax.ops.segment_min(f3, dst, n_nodes), 0.0)
    n4 = n4s / jnp.maximum(cnt, 1.0)
    vc = _mlp(p["reduce"], jnp.concatenate([nf, n1, n2, n3, n4], axis=1))
    ve = _mlp(p["ve_down"], jnp.concatenate([f1, f2, f3, f4, ef], axis=1))
    return _bn(p["bn_nf"], vc), _bn(p["bn_ef"], ve)


def _toedge_body(x_ref, w1, b1, w2, b2, w3, b3, w4, b4, w5, b5, out_ref):
    x = x_ref[...]
    h = jnp.maximum(x @ w1[...] + b1[...], 0.0)
    h = jnp.maximum(h @ w2[...] + b2[...], 0.0)
    y = h @ w3[...] + b3[...]
    k = jax.nn.sigmoid(y[:, :1])
    f = y[:, 1:65] * k
    g = jnp.maximum(f @ w4[...] + b4[...], 0.0)
    out_ref[...] = g @ w5[...] + b5[...]


def _toedge_pallas(x, p):
    w1, b1 = p["mlp1"][0]["W"], p["mlp1"][0]["b"]
    w2, b2 = p["mlp1"][1]["W"], p["mlp1"][1]["b"]
    w3, b3 = p["mlp1"][2]["W"], p["mlp1"][2]["b"]
    w4, b4 = p["mlp2"][0]["W"], p["mlp2"][0]["b"]
    w5, b5 = p["mlp2"][1]["W"], p["mlp2"][1]["b"]
    grid = E // BE
    full = lambda a: pl.BlockSpec(a.shape, lambda i: (0,) * a.ndim)
    return pl.pallas_call(
        _toedge_body,
        grid=(grid,),
        in_specs=[pl.BlockSpec((BE, x.shape[1]), lambda i: (i, 0))]
        + [full(a) for a in (w1, b1, w2, b2, w3, b3, w4, b4, w5, b5)],
        out_specs=pl.BlockSpec((BE, 1), lambda i: (i, 0)),
        out_shape=jax.ShapeDtypeStruct((E, 1), jnp.float32),
    )(x, w1, b1, w2, b2, w3, b3, w4, b4, w5, b5)


def kernel(nf, ef_cc, ef_ss, params, edge_cc, edge_ss):
    n = nf.shape[0]
    nf = nf.at[:, 1:3].set(0.0)
    ef_cc = jnp.zeros_like(ef_cc)
    ss_src, ss_dst = edge_ss[0], edge_ss[1]
    cc_src, cc_dst = edge_cc[0], edge_cc[1]
    ones_e = jnp.ones((E, 128), jnp.float32)
    ss_cnt = _segsum(ones_e, ss_dst)[:, :1]
    cc_cnt = _segsum(ones_e, cc_dst)[:, :1]
    vc0 = _lin(params["fc_ss_nf"], nf)
    ve0 = _lin(params["fc_ss_ef"], ef_ss)
    vc1, ve1 = _mp_block(params["gsmp1"], 64, vc0, ve0, ss_src, ss_dst, n, ss_cnt)
    vc2, ve2 = _mp_block(params["gsmp2"], 128, vc1, ve1, ss_src, ss_dst, n, ss_cnt)
    vc3, ve3 = _mp_block(params["gsmp3"], 64,
                         jnp.concatenate([vc2, vc1], axis=1),
                         jnp.concatenate([ve2, ve1], axis=1), ss_src, ss_dst, n, ss_cnt)
    vc4 = _mlp(params["mlp_ss_nf"], jnp.concatenate([vc3, vc0], axis=1))
    ve4 = _mlp(params["mlp_ss_ef"], jnp.concatenate([ve3, ve0], axis=1))
    del ve4
    cc_vc0 = _lin(params["fc_cc_nf"], jnp.concatenate([nf, vc4], axis=1))
    cc_ve0 = _lin(params["fc_cc_ef"], ef_cc)
    cc_vc1, cc_ve1 = _mp_block(params["gcmp1"], 64, cc_vc0, cc_ve0, cc_src, cc_dst, n, cc_cnt)
    cc_vc2, cc_ve2 = _mp_block(params["gcmp2"], 128, cc_vc1, cc_ve1, cc_src, cc_dst, n, cc_cnt)
    cc_vc3, cc_ve3 = _mp_block(params["gcmp3"], 64,
                               jnp.concatenate([cc_vc2, cc_vc1], axis=1),
                               jnp.concatenate([cc_ve2, cc_ve1], axis=1), cc_src, cc_dst, n, cc_cnt)
    nf_t = jnp.concatenate([cc_vc3, cc_vc0], axis=1)
    ef_t = jnp.concatenate([cc_ve3, cc_ve0, ef_cc], axis=1)
    x = jnp.concatenate([nf_t[cc_src], nf_t[cc_dst], ef_t], axis=1)
    return _toedge_pallas(x, params["toedge"])


# speedup vs baseline: 1.3157x; 1.0949x over previous
import functools

import jax
import jax.numpy as jnp
from jax import lax
from jax.experimental import pallas as pl
from jax.experimental.pallas import tpu as pltpu
from jax.experimental.pallas import tpu_sc as plsc

N = 10000
E = 160000
BE = 2000

NC, NS = 2, 16
NW = NC * NS
CH = 128
NCHUNK = E // CH
NPAD = 10112
RPS = NPAD // NS


def _sc_scatter_add(vals, idx):
    D = vals.shape[1]
    assert D == 128, D
    mesh = plsc.VectorSubcoreMesh(core_axis_name="c", subcore_axis_name="s")

    @functools.partial(
        pl.kernel, mesh=mesh,
        out_type=jax.ShapeDtypeStruct((NC * NPAD, D), jnp.float32),
        scratch_types=[
            pltpu.VMEM((CH,), jnp.int32),
            pltpu.VMEM((CH, D), jnp.float32),
            pltpu.VMEM_SHARED((NPAD, D), jnp.float32),
        ],
    )
    def k(vals_hbm, idx_hbm, zeros_hbm, out_hbm, idx_v, vals_v, acc):
        c = lax.axis_index("c")
        s = lax.axis_index("s")
        g = c * NS + s
        pltpu.sync_copy(zeros_hbm, acc.at[pl.ds(s * RPS, RPS)])
        plsc.subcore_barrier()
        nchunks = jnp.where(g < NCHUNK % NW, NCHUNK // NW + 1, NCHUNK // NW)
        def body(t, _):
            cid = g + t * NW
            pltpu.sync_copy(idx_hbm.at[pl.ds(cid * CH, CH)], idx_v)
            pltpu.sync_copy(vals_hbm.at[pl.ds(cid * CH, CH)], vals_v)
            pltpu.sync_copy(vals_v, acc.at[idx_v], add=True)
            return 0
        lax.fori_loop(0, nchunks, body, 0)
        plsc.subcore_barrier()
        pltpu.sync_copy(acc.at[pl.ds(s * RPS, RPS)],
                        out_hbm.at[pl.ds(c * NPAD + s * RPS, RPS)])

    return k(vals, idx, jnp.zeros((RPS, D), jnp.float32))


def _segsum(vals, idx):
    part = _sc_scatter_add(vals, idx)
    return part[:N] + part[NPAD:NPAD + N]


def _sc_gather128(table, idx):
    mesh = plsc.VectorSubcoreMesh(core_axis_name="c", subcore_axis_name="s")

    @functools.partial(
        pl.kernel, mesh=mesh,
        out_type=jax.ShapeDtypeStruct((E, 128), jnp.float32),
        scratch_types=[
            pltpu.VMEM((CH,), jnp.int32),
            pltpu.VMEM((CH, 128), jnp.float32),
            pltpu.SemaphoreType.DMA,
        ],
    )
    def k(table_hbm, idx_hbm, out_hbm, idx_v, rows_v, sem):
        c = lax.axis_index("c")
        s = lax.axis_index("s")
        g = c * NS + s
        nchunks = jnp.where(g < NCHUNK % NW, NCHUNK // NW + 1, NCHUNK // NW)

        def body(t, _):
            cid = g + t * NW
            pltpu.sync_copy(idx_hbm.at[pl.ds(cid * CH, CH)], idx_v)
            pltpu.async_copy(table_hbm.at[idx_v], rows_v, sem).wait()
            pltpu.sync_copy(rows_v, out_hbm.at[pl.ds(cid * CH, CH)])
            return 0

        lax.fori_loop(0, nchunks, body, 0)

    return k(table, idx)


def _gather_rows(tab, idx):
    d = tab.shape[1]
    outs = []
    for c0 in range(0, d, 128):
        part = tab[:, c0:c0 + 128]
        pd = part.shape[1]
        if pd < 128:
            part = jnp.pad(part, ((0, 0), (0, 128 - pd)))
        outs.append(_sc_gather128(part, idx)[:, :pd])
    return jnp.concatenate(outs, axis=1) if len(outs) > 1 else outs[0]


def _lin(p, x):
    return x @ p["W"] + p["b"]


def _mlp(ps, x):
    for i, p in enumerate(ps):
        x = _lin(p, x)
        if i < len(ps) - 1:
            x = jax.nn.relu(x)
    return x


def _bn(p, x, eps=1e-5):
    m = jnp.mean(x, axis=0)
    v = jnp.var(x, axis=0)
    return (x - m) / jnp.sqrt(v + eps) * p["gamma"] + p["beta"]


def _mp_block(p, n_out, nf, ef, src, dst, n_nodes, cnt):
    x = jnp.concatenate([_gather_rows(nf, src), _gather_rows(nf, dst), ef], axis=1)
    x = _mlp(p["msg"], x)
    k = jax.nn.sigmoid(x[:, :1])
    f1 = x[:, 1:1 + n_out] * k
    f2 = x[:, 1 + n_out:1 + 2 * n_out] * k
    f3 = x[:, 1 + 2 * n_out:1 + 3 * n_out] * k
    f4 = x[:, 1 + 3 * n_out:1 + 4 * n_out] * k
    has = cnt > 0
    if n_out == 64:
        s14 = _segsum(jnp.concatenate([f1, f4], axis=1), dst)
        n1, n4s = s14[:, :64], s14[:, 64:]
    else:
        n1 = _segsum(f1, dst)
        n4s = _segsum(f4, dst)
    n2 = jnp.where(has, jax.ops.segment_max(f2, dst, n_nodes), 0.0)
    n3 = jnp.where(has, jax.ops.segment_min(f3, dst, n_nodes), 0.0)
    n4 = n4s / jnp.maximum(cnt, 1.0)
    vc = _mlp(p["reduce"], jnp.concatenate([nf, n1, n2, n3, n4], axis=1))
    ve = _mlp(p["ve_down"], jnp.concatenate([f1, f2, f3, f4, ef], axis=1))
    return _bn(p["bn_nf"], vc), _bn(p["bn_ef"], ve)


def _toedge_body(x_ref, w1, b1, w2, b2, w3, b3, w4, b4, w5, b5, out_ref):
    x = x_ref[...]
    h = jnp.maximum(x @ w1[...] + b1[...], 0.0)
    h = jnp.maximum(h @ w2[...] + b2[...], 0.0)
    y = h @ w3[...] + b3[...]
    k = jax.nn.sigmoid(y[:, :1])
    f = y[:, 1:65] * k
    g = jnp.maximum(f @ w4[...] + b4[...], 0.0)
    out_ref[...] = g @ w5[...] + b5[...]


def _toedge_pallas(x, p):
    w1, b1 = p["mlp1"][0]["W"], p["mlp1"][0]["b"]
    w2, b2 = p["mlp1"][1]["W"], p["mlp1"][1]["b"]
    w3, b3 = p["mlp1"][2]["W"], p["mlp1"][2]["b"]
    w4, b4 = p["mlp2"][0]["W"], p["mlp2"][0]["b"]
    w5, b5 = p["mlp2"][1]["W"], p["mlp2"][1]["b"]
    grid = E // BE
    full = lambda a: pl.BlockSpec(a.shape, lambda i: (0,) * a.ndim)
    return pl.pallas_call(
        _toedge_body,
        grid=(grid,),
        in_specs=[pl.BlockSpec((BE, x.shape[1]), lambda i: (i, 0))]
        + [full(a) for a in (w1, b1, w2, b2, w3, b3, w4, b4, w5, b5)],
        out_specs=pl.BlockSpec((BE, 1), lambda i: (i, 0)),
        out_shape=jax.ShapeDtypeStruct((E, 1), jnp.float32),
    )(x, w1, b1, w2, b2, w3, b3, w4, b4, w5, b5)


def kernel(nf, ef_cc, ef_ss, params, edge_cc, edge_ss):
    n = nf.shape[0]
    nf = nf.at[:, 1:3].set(0.0)
    ef_cc = jnp.zeros_like(ef_cc)
    ss_src, ss_dst = edge_ss[0], edge_ss[1]
    cc_src, cc_dst = edge_cc[0], edge_cc[1]
    ones_e = jnp.ones((E, 128), jnp.float32)
    ss_cnt = _segsum(ones_e, ss_dst)[:, :1]
    cc_cnt = _segsum(ones_e, cc_dst)[:, :1]
    vc0 = _lin(params["fc_ss_nf"], nf)
    ve0 = _lin(params["fc_ss_ef"], ef_ss)
    vc1, ve1 = _mp_block(params["gsmp1"], 64, vc0, ve0, ss_src, ss_dst, n, ss_cnt)
    vc2, ve2 = _mp_block(params["gsmp2"], 128, vc1, ve1, ss_src, ss_dst, n, ss_cnt)
    vc3, ve3 = _mp_block(params["gsmp3"], 64,
                         jnp.concatenate([vc2, vc1], axis=1),
                         jnp.concatenate([ve2, ve1], axis=1), ss_src, ss_dst, n, ss_cnt)
    vc4 = _mlp(params["mlp_ss_nf"], jnp.concatenate([vc3, vc0], axis=1))
    ve4 = _mlp(params["mlp_ss_ef"], jnp.concatenate([ve3, ve0], axis=1))
    del ve4
    cc_vc0 = _lin(params["fc_cc_nf"], jnp.concatenate([nf, vc4], axis=1))
    cc_ve0 = _lin(params["fc_cc_ef"], ef_cc)
    cc_vc1, cc_ve1 = _mp_block(params["gcmp1"], 64, cc_vc0, cc_ve0, cc_src, cc_dst, n, cc_cnt)
    cc_vc2, cc_ve2 = _mp_block(params["gcmp2"], 128, cc_vc1, cc_ve1, cc_src, cc_dst, n, cc_cnt)
    cc_vc3, cc_ve3 = _mp_block(params["gcmp3"], 64,
                               jnp.concatenate([cc_vc2, cc_vc1], axis=1),
                               jnp.concatenate([cc_ve2, cc_ve1], axis=1), cc_src, cc_dst, n, cc_cnt)
    nf_t = jnp.concatenate([cc_vc3, cc_vc0], axis=1)
    ef_t = jnp.concatenate([cc_ve3, cc_ve0, ef_cc], axis=1)
    x = jnp.concatenate([_gather_rows(nf_t, cc_src), _gather_rows(nf_t, cc_dst), ef_t], axis=1)
    return _toedge_pallas(x, params["toedge"])
